# baseline (speedup 1.0000x reference)
import functools

import jax
import jax.numpy as jnp
from jax.experimental import pallas as pl


def _ln(x, g, b, eps=1e-5):
    m = jnp.mean(x, axis=-1, keepdims=True)
    v = jnp.mean((x - m) ** 2, axis=-1, keepdims=True)
    return (x - m) / jnp.sqrt(v + eps) * g + b


def _fused_kernel(x_ref, pred_ln_g, pred_ln_b, pred_w1, pred_b1, pred_w2,
                  pred_b2, ln2_g, ln2_b, mlp_w1, mlp_b1, mlp_w2, mlp_b2,
                  fast_ln_g, fast_ln_b, fast_w1, fast_b1, fast_w2, fast_b2,
                  out_ref, *, num_keep):
    x = x_ref[0]
    N = x.shape[0]

    s = _ln(x, pred_ln_g[...], pred_ln_b[...])
    s = jax.nn.gelu(jnp.dot(s, pred_w1[...]) + pred_b1[...])
    logits = jnp.dot(s, pred_w2[...]) + pred_b2[...]
    m = jnp.max(logits, axis=-1, keepdims=True)
    e = jnp.exp(logits - m)
    score = e[:, 0:1] / jnp.sum(e, axis=-1, keepdims=True)

    keys = jax.lax.bitcast_convert_type(score, jnp.int32)
    k = jnp.int32(num_keep)

    def body_val(_, c):
        lo, hi = c
        mid = (lo + hi) // 2
        ge = jnp.sum((keys >= mid).astype(jnp.int32)) >= k
        return jnp.where(ge, mid, lo), jnp.where(ge, hi, mid)

    v, _ = jax.lax.fori_loop(
        0, 31, body_val, (jnp.int32(0), jnp.int32(0x3F800001)))

    n_gt = jnp.sum((keys > v).astype(jnp.int32))
    r = k - n_gt
    tie = keys == v
    idx = jax.lax.broadcasted_iota(jnp.int32, (N, 1), 0)

    def body_idx(_, c):
        lo, hi = c
        mid = (lo + hi) // 2
        ok = jnp.sum((tie & (idx < mid)).astype(jnp.int32)) >= r
        return jnp.where(ok, lo, mid + 1), jnp.where(ok, mid, hi)

    t_idx, _ = jax.lax.fori_loop(
        0, 11, body_idx, (jnp.int32(0), jnp.int32(N)))

    keep = (keys > v) | (tie & (idx < t_idx))

    h = _ln(x, ln2_g[...], ln2_b[...])
    h = jnp.dot(jax.nn.gelu(jnp.dot(h, mlp_w1[...]) + mlp_b1[...]),
                mlp_w2[...]) + mlp_b2[...]

    h2 = _ln(x, fast_ln_g[...], fast_ln_b[...])
    h2 = jnp.dot(jax.nn.gelu(jnp.dot(h2, fast_w1[...]) + fast_b1[...]),
                 fast_w2[...]) + fast_b2[...]

    out_ref[0] = x + jnp.where(keep, h, h2)


def kernel(x, pred_ln_g, pred_ln_b, pred_w1, pred_b1, pred_w2, pred_b2,
           ln2_g, ln2_b, mlp_w1, mlp_b1, mlp_w2, mlp_b2,
           fast_ln_g, fast_ln_b, fast_w1, fast_b1, fast_w2, fast_b2):
    B, N, C = x.shape
    num_keep = N // 2

    vecs = dict(
        pred_ln_g=pred_ln_g, pred_ln_b=pred_ln_b, pred_b1=pred_b1,
        pred_b2=pred_b2, ln2_g=ln2_g, ln2_b=ln2_b, mlp_b1=mlp_b1,
        mlp_b2=mlp_b2, fast_ln_g=fast_ln_g, fast_ln_b=fast_ln_b,
        fast_b1=fast_b1, fast_b2=fast_b2,
    )
    vecs = {k: v.reshape(1, -1) for k, v in vecs.items()}

    def full(a):
        return pl.BlockSpec(a.shape, lambda b: (0,) * a.ndim)

    args = (x, vecs['pred_ln_g'], vecs['pred_ln_b'], pred_w1, vecs['pred_b1'],
            pred_w2, vecs['pred_b2'], vecs['ln2_g'], vecs['ln2_b'],
            mlp_w1, vecs['mlp_b1'], mlp_w2, vecs['mlp_b2'],
            vecs['fast_ln_g'], vecs['fast_ln_b'], fast_w1, vecs['fast_b1'],
            fast_w2, vecs['fast_b2'])

    in_specs = [pl.BlockSpec((1, N, C), lambda b: (b, 0, 0))]
    in_specs += [full(a) for a in args[1:]]

    return pl.pallas_call(
        functools.partial(_fused_kernel, num_keep=num_keep),
        grid=(B,),
        in_specs=in_specs,
        out_specs=pl.BlockSpec((1, N, C), lambda b: (b, 0, 0)),
        out_shape=jax.ShapeDtypeStruct((B, N, C), x.dtype),
    )(*args)


# speedup vs baseline: 3.2741x; 3.2741x over previous
import functools

import jax
import jax.numpy as jnp
from jax.experimental import pallas as pl


def _ln(x, g, b, eps=1e-5):
    m = jnp.mean(x, axis=-1, keepdims=True)
    v = jnp.mean((x - m) ** 2, axis=-1, keepdims=True)
    return (x - m) / jnp.sqrt(v + eps) * g + b


def _score_kernel(x_ref, g_ref, b_ref, w1_ref, b1_ref, w2_ref, b2_ref,
                  score_ref):
    s = _ln(x_ref[...], g_ref[...], b_ref[...])
    s = jax.nn.gelu(jnp.dot(s, w1_ref[...]) + b1_ref[...])
    logits = jnp.dot(s, w2_ref[...]) + b2_ref[...]
    m = jnp.max(logits, axis=-1, keepdims=True)
    e = jnp.exp(logits - m)
    score_ref[...] = e[:, 0:1] / jnp.sum(e, axis=-1, keepdims=True)


def _mask_kernel(score_ref, mask_ref, *, num_keep):
    B, N = score_ref.shape
    keys = jax.lax.bitcast_convert_type(score_ref[...], jnp.int32)
    k = jnp.int32(num_keep)

    def count_ge(t):
        return jnp.sum((keys >= t).astype(jnp.int32), axis=1, keepdims=True)

    def body_val(_, c):
        lo, hi = c
        mid = (lo + hi) // 2
        ge = count_ge(mid) >= k
        return jnp.where(ge, mid, lo), jnp.where(ge, hi, mid)

    lo0 = jnp.zeros((B, 1), jnp.int32)
    hi0 = jnp.full((B, 1), 0x3F800001, jnp.int32)
    v, _ = jax.lax.fori_loop(0, 31, body_val, (lo0, hi0))

    n_gt = jnp.sum((keys > v).astype(jnp.int32), axis=1, keepdims=True)
    r = k - n_gt
    tie = keys == v
    idx = jax.lax.broadcasted_iota(jnp.int32, (B, N), 1)

    def body_idx(_, c):
        lo, hi = c
        mid = (lo + hi) // 2
        cnt = jnp.sum((tie & (idx < mid)).astype(jnp.int32), axis=1,
                      keepdims=True)
        ok = cnt >= r
        return jnp.where(ok, lo, mid + 1), jnp.where(ok, mid, hi)

    t_idx, _ = jax.lax.fori_loop(
        0, 11, body_idx, (jnp.zeros((B, 1), jnp.int32),
                          jnp.full((B, 1), N, jnp.int32)))

    keep = (keys > v) | (tie & (idx < t_idx))
    mask_ref[...] = keep.astype(jnp.float32)


def _mlp_kernel(x_ref, mask_ref, ln2_g, ln2_b, mlp_w1, mlp_b1, mlp_w2,
                mlp_b2, fast_ln_g, fast_ln_b, fast_w1, fast_b1, fast_w2,
                fast_b2, out_ref):
    x = x_ref[...]
    h = _ln(x, ln2_g[...], ln2_b[...])
    h = jnp.dot(jax.nn.gelu(jnp.dot(h, mlp_w1[...]) + mlp_b1[...]),
                mlp_w2[...]) + mlp_b2[...]
    h2 = _ln(x, fast_ln_g[...], fast_ln_b[...])
    h2 = jnp.dot(jax.nn.gelu(jnp.dot(h2, fast_w1[...]) + fast_b1[...]),
                 fast_w2[...]) + fast_b2[...]
    out_ref[...] = x + jnp.where(mask_ref[...] > 0.5, h, h2)


def _full(a):
    return pl.BlockSpec(a.shape, lambda i: (0,) * a.ndim)


def kernel(x, pred_ln_g, pred_ln_b, pred_w1, pred_b1, pred_w2, pred_b2,
           ln2_g, ln2_b, mlp_w1, mlp_b1, mlp_w2, mlp_b2,
           fast_ln_g, fast_ln_b, fast_w1, fast_b1, fast_w2, fast_b2):
    B, N, C = x.shape
    num_keep = N // 2
    M = B * N
    xf = x.reshape(M, C)

    r2 = lambda a: a.reshape(1, -1)

    T1 = 8192
    scores = pl.pallas_call(
        _score_kernel,
        grid=(M // T1,),
        in_specs=[pl.BlockSpec((T1, C), lambda i: (i, 0)),
                  _full(r2(pred_ln_g)), _full(r2(pred_ln_b)),
                  _full(pred_w1), _full(r2(pred_b1)),
                  _full(pred_w2), _full(r2(pred_b2))],
        out_specs=pl.BlockSpec((T1, 1), lambda i: (i, 0)),
        out_shape=jax.ShapeDtypeStruct((M, 1), jnp.float32),
    )(xf, r2(pred_ln_g), r2(pred_ln_b), pred_w1, r2(pred_b1),
      pred_w2, r2(pred_b2))

    mask = pl.pallas_call(
        functools.partial(_mask_kernel, num_keep=num_keep),
        in_specs=[pl.BlockSpec((B, N), lambda: (0, 0))],
        out_specs=pl.BlockSpec((B, N), lambda: (0, 0)),
        out_shape=jax.ShapeDtypeStruct((B, N), jnp.float32),
    )(scores.reshape(B, N))

    T3 = 4096
    out = pl.pallas_call(
        _mlp_kernel,
        grid=(M // T3,),
        in_specs=[pl.BlockSpec((T3, C), lambda i: (i, 0)),
                  pl.BlockSpec((T3, 1), lambda i: (i, 0)),
                  _full(r2(ln2_g)), _full(r2(ln2_b)),
                  _full(mlp_w1), _full(r2(mlp_b1)),
                  _full(mlp_w2), _full(r2(mlp_b2)),
                  _full(r2(fast_ln_g)), _full(r2(fast_ln_b)),
                  _full(fast_w1), _full(r2(fast_b1)),
                  _full(fast_w2), _full(r2(fast_b2))],
        out_specs=pl.BlockSpec((T3, C), lambda i: (i, 0)),
        out_shape=jax.ShapeDtypeStruct((M, C), x.dtype),
    )(xf, mask.reshape(M, 1), r2(ln2_g), r2(ln2_b), mlp_w1, r2(mlp_b1),
      mlp_w2, r2(mlp_b2), r2(fast_ln_g), r2(fast_ln_b), fast_w1,
      r2(fast_b1), fast_w2, r2(fast_b2))

    return out.reshape(B, N, C)
